# baseline (speedup 1.0000x reference)
import functools

import jax
import jax.numpy as jnp
from jax import lax
from jax.experimental import pallas as pl
from jax.experimental.pallas import tpu as pltpu
from jax.experimental.pallas import tpu_sc as plsc

B, L, DIM, S = 1024, 20, 64, 12
N_NODES = 1000000
PAD = 128
NW = 32
N_ITEMS = B * L
N_NBR = B * L * S
CHUNK = 128


EMB_C = 4096


def _emb_repack_body(in_ref, out_ref):
    x = in_ref[...]
    eye = (lax.broadcasted_iota(jnp.int32, (DIM, DIM), 0)
           == lax.broadcasted_iota(jnp.int32, (DIM, DIM), 1)).astype(jnp.float32)
    y = lax.dot_general(x, eye, (((0,), (0,)), ((), ())),
                        preferred_element_type=jnp.float32)
    y3 = y.reshape(EMB_C // 2, 2, DIM)
    out_ref[...] = jnp.concatenate([y3[:, 0, :], y3[:, 1, :]], axis=1)


def _emb_repack(emb_t):
    n = emb_t.shape[1]
    grid = (n + EMB_C - 1) // EMB_C
    return pl.pallas_call(
        _emb_repack_body,
        grid=(grid,),
        in_specs=[pl.BlockSpec((DIM, EMB_C), lambda i: (0, i))],
        out_specs=pl.BlockSpec((EMB_C // 2, 2 * DIM), lambda i: (i, 0)),
        out_shape=jax.ShapeDtypeStruct((n // 2, 2 * DIM), jnp.float32),
    )(emb_t)


def _worker_id():
    return lax.axis_index("s") * 2 + lax.axis_index("c")


def _stage1(items_flat, input_flat, emb_lin):
    per = N_ITEMS // NW
    nchunk = per // CHUNK
    mesh = plsc.VectorSubcoreMesh(core_axis_name="c", subcore_axis_name="s")

    @functools.partial(
        pl.kernel,
        out_type=(
            jax.ShapeDtypeStruct((N_ITEMS, PAD), jnp.float32),
            jax.ShapeDtypeStruct((N_ITEMS, PAD), jnp.float32),
        ),
        mesh=mesh,
        compiler_params=pltpu.CompilerParams(use_tc_tiling_on_sc=False, needs_layout_passes=False),
        scratch_types=[
            pltpu.VMEM((per,), jnp.int32),
            pltpu.VMEM((per,), jnp.int32),
            pltpu.VMEM((per, DIM), jnp.float32),
            pltpu.VMEM((per, DIM), jnp.float32),
            pltpu.SemaphoreType.DMA,
        ],
    )
    def k(items_hbm, input_hbm, emb_hbm,
          h_out, ie_out,
          idx_v, idx2_v, h_v, ie_v, sem):
        wid = _worker_id()
        base = pl.multiple_of(wid * per, 8)
        pltpu.sync_copy(items_hbm.at[pl.ds(base, per)], idx_v)
        pltpu.sync_copy(input_hbm.at[pl.ds(base, per)], idx2_v)
        copies = []
        for j in range(nchunk):
            sl = pl.ds(j * CHUNK, CHUNK)
            copies.append(pltpu.async_copy(emb_hbm.at[idx_v.at[sl]], h_v.at[sl], sem))
            copies.append(pltpu.async_copy(emb_hbm.at[idx2_v.at[sl]], ie_v.at[sl], sem))
        for c in copies:
            c.wait()
        osl = pl.ds(base, per)
        csl = pl.ds(0, DIM)
        pltpu.sync_copy(h_v, h_out.at[osl, csl])
        pltpu.sync_copy(ie_v, ie_out.at[osl, csl])

    return k(items_flat, input_flat, emb_lin)


def _stage2(nbr_idx2d, nbr_w2d, emb_lin):
    per = N_NBR // NW
    ni_per = N_ITEMS // NW
    rows_per_buf = 256
    nchunk = rows_per_buf // CHUNK
    nbuf_steps = per // rows_per_buf
    mesh = plsc.VectorSubcoreMesh(core_axis_name="c", subcore_axis_name="s")

    @functools.partial(
        pl.kernel,
        out_type=jax.ShapeDtypeStruct((N_NBR, PAD), jnp.float32),
        mesh=mesh,
        compiler_params=pltpu.CompilerParams(use_tc_tiling_on_sc=False, needs_layout_passes=False),
        scratch_types=[
            pltpu.VMEM((ni_per, S), jnp.int32),
            pltpu.VMEM((ni_per, S), jnp.float32),
            pltpu.VMEM((per,), jnp.int32),
            pltpu.VMEM((per, 1), jnp.float32),
            pltpu.VMEM((rows_per_buf, DIM), jnp.float32),
            pltpu.VMEM((rows_per_buf, DIM), jnp.float32),
            pltpu.SemaphoreType.DMA,
            pltpu.SemaphoreType.DMA,
            pltpu.SemaphoreType.DMA,
        ],
    )
    def k(idx_hbm, w_hbm, emb_hbm, out_hbm, idx2d_v, w2d_v, idx_v, w_v, b0, b1, s0, s1, sw):
        wid = _worker_id()
        base = pl.multiple_of(wid * per, 8)
        isl = pl.ds(pl.multiple_of(wid * ni_per, 8), ni_per)
        pltpu.sync_copy(idx_hbm.at[isl], idx2d_v)
        pltpu.sync_copy(w_hbm.at[isl], w2d_v)

        lane = jnp.arange(16, dtype=jnp.int32)

        def flatten_body(v, carry):
            fl = lane + v * 16
            ri = (fl * 2731) >> 15
            ci = fl - ri * S
            plsc.store_scatter(idx_v, [fl], plsc.load_gather(idx2d_v, [ri, ci]))
            plsc.store_scatter(w_v, [fl, fl - fl], plsc.load_gather(w2d_v, [ri, ci]))
            return carry

        lax.fori_loop(0, per // 16, flatten_body, 0)
        csl = pl.ds(0, DIM)
        wcopy = pltpu.async_copy(w_v, out_hbm.at[pl.ds(base, per), pl.ds(DIM, 1)], sw)

        def body(i, carry):
            t0 = 2 * i
            t1 = 2 * i + 1
            o0 = pl.multiple_of(t0 * rows_per_buf, 8)
            o1 = pl.multiple_of(t1 * rows_per_buf, 8)
            c0 = []
            c1 = []
            for j in range(nchunk):
                sl = pl.ds(j * CHUNK, CHUNK)
                c0.append(pltpu.async_copy(
                    emb_hbm.at[idx_v.at[pl.ds(o0 + j * CHUNK, CHUNK)]], b0.at[sl], s0))
            for j in range(nchunk):
                sl = pl.ds(j * CHUNK, CHUNK)
                c1.append(pltpu.async_copy(
                    emb_hbm.at[idx_v.at[pl.ds(o1 + j * CHUNK, CHUNK)]], b1.at[sl], s1))
            for c in c0:
                c.wait()
            r0 = pl.ds(pl.multiple_of(base + o0, 8), rows_per_buf)
            pltpu.sync_copy(b0, out_hbm.at[r0, csl])
            for c in c1:
                c.wait()
            r1 = pl.ds(pl.multiple_of(base + o1, 8), rows_per_buf)
            pltpu.sync_copy(b1, out_hbm.at[r1, csl])
            return carry

        lax.fori_loop(0, nbuf_steps // 2, body, 0)
        wcopy.wait()

    return k(nbr_idx2d, nbr_w2d, emb_lin)


def _leaky(x):
    return jnp.where(x >= 0, x, 0.2 * x)


BB = 16


def _tc_body(h_ref, ie_ref, nv_ref, adj_ref, mf_ref, al_ref,
             a4_ref, w1a_ref, w1b_ref, w2_ref, w3a_ref, w3b_ref, bias_ref,
             out_ref):
    a4 = a4_ref[...]
    BL = BB * L
    h2 = h_ref[...][:, :DIM]

    neg = jnp.full((L, L), -9e15, jnp.float32)
    att_rows = []
    for b in range(BB):
        hb = h2[b * L:(b + 1) * L]
        adjb = adj_ref[b]
        m4 = (a4[:, None, :] * hb[None, :, :]).reshape(4 * L, DIM)
        e_all = _leaky(lax.dot_general(m4, hb, (((1,), (1,)), ((), ())),
                                       preferred_element_type=jnp.float32))
        att = jnp.where(adjb == 1, e_all[0:L], neg)
        att = jnp.where(adjb == 2, e_all[L:2 * L], att)
        att = jnp.where(adjb == 3, e_all[2 * L:3 * L], att)
        att = jnp.where(adjb == 4, e_all[3 * L:4 * L], att)
        mx = jnp.max(att, axis=-1, keepdims=True)
        p = jnp.exp(att - mx)
        att = p / jnp.sum(p, axis=-1, keepdims=True)
        if b == 0:
            row = jnp.concatenate([att, jnp.zeros((L, BL - L), jnp.float32)], axis=1)
        elif b == BB - 1:
            row = jnp.concatenate([jnp.zeros((L, BL - L), jnp.float32), att], axis=1)
        else:
            row = jnp.concatenate([jnp.zeros((L, b * L), jnp.float32), att,
                                   jnp.zeros((L, BL - (b + 1) * L), jnp.float32)], axis=1)
        att_rows.append(row)
    att_bd = jnp.concatenate(att_rows, axis=0)
    h_local = jnp.dot(att_bd, h2, preferred_element_type=jnp.float32)

    mf = mf_ref[...]
    ie3 = ie_ref[...][:, :DIM].reshape(BB, L, DIM)
    sess = jnp.sum(mf.reshape(BB, L, 1) * ie3, axis=1)
    sess = sess / jnp.sum(mf, axis=-1)

    nv_blk = nv_ref[...]
    nv = nv_blk[:, :DIM]
    q = (nv.reshape(BB, L * S, DIM) * sess[:, None, :]).reshape(BB * L * S, DIM)
    nw2 = nv_blk[:, DIM:DIM + 1]
    t = _leaky(jnp.dot(q, w1a_ref[...], preferred_element_type=jnp.float32)
               + nw2 * w1b_ref[...])
    t3 = t.reshape(BL, S, DIM)
    nv3 = nv.reshape(BL, S, DIM)
    w2 = w2_ref[...]
    ag_cols = [jnp.sum(t3[:, s, :] * w2, axis=-1, keepdims=True)
               for s in range(S)]
    ag = jnp.concatenate(ag_cols, axis=1)
    mx2 = jnp.max(ag, axis=-1, keepdims=True)
    p2 = jnp.exp(ag - mx2)
    ag = p2 / jnp.sum(p2, axis=-1, keepdims=True)
    nbr_agg = ag[:, 0:1] * nv3[:, 0, :]
    for s in range(1, S):
        nbr_agg = nbr_agg + ag[:, s:s + 1] * nv3[:, s, :]

    out = (jnp.dot(h2, w3a_ref[...], preferred_element_type=jnp.float32)
           + jnp.dot(nbr_agg, w3b_ref[...], preferred_element_type=jnp.float32)
           + bias_ref[...])
    h_global = jnp.maximum(out, 0.0)
    comb = h_local + h_global

    al2 = al_ref[...].reshape(BL, 1)
    rows = lax.broadcasted_iota(jnp.int32, (BL, 1), 0)
    tgt = al2 + (rows // L) * L
    oh = (tgt == lax.broadcasted_iota(jnp.int32, (BL, BL), 1)).astype(jnp.float32)
    final = jnp.dot(oh, comb, preferred_element_type=jnp.float32)
    out_ref[...] = final.reshape(BB, L, DIM)


def _tc_compute(h, ie, nv, adj, mf, al, a4, w1a, w1b, w2, w3a, w3b, bias,
                interpret=False):
    row2 = lambda i: (i, 0)
    row3 = lambda i: (i, 0, 0)
    zero2 = lambda i: (0, 0)
    return pl.pallas_call(
        _tc_body,
        grid=(B // BB,),
        in_specs=[
            pl.BlockSpec((BB * L, PAD), row2),
            pl.BlockSpec((BB * L, PAD), row2),
            pl.BlockSpec((BB * L * S, PAD), row2),
            pl.BlockSpec((BB, L, L), row3),
            pl.BlockSpec((BB, 1, L), row3),
            pl.BlockSpec((BB, L, 1), row3),
            pl.BlockSpec((4, DIM), zero2),
            pl.BlockSpec((DIM, DIM), zero2),
            pl.BlockSpec((1, DIM), zero2),
            pl.BlockSpec((1, DIM), zero2),
            pl.BlockSpec((DIM, DIM), zero2),
            pl.BlockSpec((DIM, DIM), zero2),
            pl.BlockSpec((1, DIM), zero2),
        ],
        out_specs=pl.BlockSpec((BB, L, DIM), row3),
        out_shape=jax.ShapeDtypeStruct((B, L, DIM), jnp.float32),
        interpret=interpret,
    )(h, ie, nv, adj, mf, al, a4, w1a, w1b, w2, w3a, w3b, bias)


def kernel(input_seq, mask, items_seq, adj_seq, alias_seq, emb_table,
           adj_global, num_global, a0, a1, a2, a3, g_w1, g_w2, g_w3, g_bias):
    idx = items_seq.reshape(N_ITEMS)
    emb_lin = _emb_repack(emb_table.T).reshape(N_NODES, DIM)
    h_pad, ie_pad = _stage1(idx, input_seq.reshape(N_ITEMS), emb_lin)
    nbr_items = jnp.take(adj_global, idx, axis=0, mode="clip")
    nbr_w = jnp.take(num_global, idx, axis=0, mode="clip")
    nv_pad = _stage2(nbr_items, nbr_w, emb_lin)

    mf = mask.astype(jnp.float32).reshape(B, 1, L)
    al = alias_seq.reshape(B, L, 1)
    a4 = jnp.concatenate([a0, a1, a2, a3], axis=1).T
    w1a = g_w1[:DIM]
    w1b = g_w1[DIM:DIM + 1]
    w2 = g_w2.T
    w3a = g_w3[:DIM]
    w3b = g_w3[DIM:]
    bias = g_bias.reshape(1, DIM)
    return _tc_compute(h_pad, ie_pad, nv_pad, adj_seq, mf, al,
                       a4, w1a, w1b, w2, w3a, w3b, bias)


# speedup vs baseline: 1.1833x; 1.1833x over previous
import functools

import jax
import jax.numpy as jnp
from jax import lax
from jax.experimental import pallas as pl
from jax.experimental.pallas import tpu as pltpu
from jax.experimental.pallas import tpu_sc as plsc

B, L, DIM, S = 1024, 20, 64, 12
N_NODES = 1000000
PAD = 128
NW = 32
N_ITEMS = B * L
N_NBR = B * L * S
CHUNK = 128


EMB_C = 4096


def _emb_repack_body(in_ref, out_ref):
    x = in_ref[...]
    eye = (lax.broadcasted_iota(jnp.int32, (DIM, DIM), 0)
           == lax.broadcasted_iota(jnp.int32, (DIM, DIM), 1)).astype(jnp.float32)
    y = lax.dot_general(x, eye, (((0,), (0,)), ((), ())),
                        preferred_element_type=jnp.float32)
    y3 = y.reshape(EMB_C // 2, 2, DIM)
    out_ref[...] = jnp.concatenate([y3[:, 0, :], y3[:, 1, :]], axis=1)


def _emb_repack(emb_t):
    n = emb_t.shape[1]
    grid = (n + EMB_C - 1) // EMB_C
    return pl.pallas_call(
        _emb_repack_body,
        grid=(grid,),
        in_specs=[pl.BlockSpec((DIM, EMB_C), lambda i: (0, i))],
        out_specs=pl.BlockSpec((EMB_C // 2, 2 * DIM), lambda i: (i, 0)),
        out_shape=jax.ShapeDtypeStruct((n // 2, 2 * DIM), jnp.float32),
    )(emb_t)


def _worker_id():
    return lax.axis_index("s") * 2 + lax.axis_index("c")


def _stage1(items_flat, input_flat, emb_lin):
    per = N_ITEMS // NW
    nchunk = per // CHUNK
    mesh = plsc.VectorSubcoreMesh(core_axis_name="c", subcore_axis_name="s")

    @functools.partial(
        pl.kernel,
        out_type=(
            jax.ShapeDtypeStruct((N_ITEMS, PAD), jnp.float32),
            jax.ShapeDtypeStruct((N_ITEMS, PAD), jnp.float32),
        ),
        mesh=mesh,
        compiler_params=pltpu.CompilerParams(use_tc_tiling_on_sc=False, needs_layout_passes=False),
        scratch_types=[
            pltpu.VMEM((per,), jnp.int32),
            pltpu.VMEM((per,), jnp.int32),
            pltpu.VMEM((per, DIM), jnp.float32),
            pltpu.VMEM((per, DIM), jnp.float32),
            pltpu.SemaphoreType.DMA,
        ],
    )
    def k(items_hbm, input_hbm, emb_hbm,
          h_out, ie_out,
          idx_v, idx2_v, h_v, ie_v, sem):
        wid = _worker_id()
        base = pl.multiple_of(wid * per, 8)
        pltpu.sync_copy(items_hbm.at[pl.ds(base, per)], idx_v)
        pltpu.sync_copy(input_hbm.at[pl.ds(base, per)], idx2_v)
        copies = []
        for j in range(nchunk):
            sl = pl.ds(j * CHUNK, CHUNK)
            copies.append(pltpu.async_copy(emb_hbm.at[idx_v.at[sl]], h_v.at[sl], sem))
            copies.append(pltpu.async_copy(emb_hbm.at[idx2_v.at[sl]], ie_v.at[sl], sem))
        for c in copies:
            c.wait()
        osl = pl.ds(base, per)
        csl = pl.ds(0, DIM)
        pltpu.sync_copy(h_v, h_out.at[osl, csl])
        pltpu.sync_copy(ie_v, ie_out.at[osl, csl])

    return k(items_flat, input_flat, emb_lin)


def _stage2(nbr_idx2d, emb_lin):
    per = N_NBR // NW
    ni_per = N_ITEMS // NW
    rows_per_buf = 640
    nchunk = rows_per_buf // CHUNK
    nbuf_steps = per // rows_per_buf
    mesh = plsc.VectorSubcoreMesh(core_axis_name="c", subcore_axis_name="s")

    @functools.partial(
        pl.kernel,
        out_type=jax.ShapeDtypeStruct((N_NBR, PAD), jnp.float32),
        mesh=mesh,
        compiler_params=pltpu.CompilerParams(use_tc_tiling_on_sc=False, needs_layout_passes=False),
        scratch_types=[
            pltpu.VMEM((ni_per, S), jnp.int32),
            pltpu.VMEM((per,), jnp.int32),
            pltpu.VMEM((rows_per_buf, DIM), jnp.float32),
            pltpu.VMEM((rows_per_buf, DIM), jnp.float32),
            pltpu.SemaphoreType.DMA,
            pltpu.SemaphoreType.DMA,
        ],
    )
    def k(idx_hbm, emb_hbm, out_hbm, idx2d_v, idx_v, b0, b1, s0, s1):
        wid = _worker_id()
        base = pl.multiple_of(wid * per, 8)
        isl = pl.ds(pl.multiple_of(wid * ni_per, 8), ni_per)
        pltpu.sync_copy(idx_hbm.at[isl], idx2d_v)

        lane = jnp.arange(16, dtype=jnp.int32)

        def flatten_body(v, carry):
            fl = lane + v * 16
            ri = (fl * 2731) >> 15
            ci = fl - ri * S
            plsc.store_scatter(idx_v, [fl], plsc.load_gather(idx2d_v, [ri, ci]))
            return carry

        lax.fori_loop(0, per // 16, flatten_body, 0)
        csl = pl.ds(0, DIM)

        def body(i, carry):
            t0 = 2 * i
            t1 = 2 * i + 1
            o0 = pl.multiple_of(t0 * rows_per_buf, 8)
            o1 = pl.multiple_of(t1 * rows_per_buf, 8)
            c0 = []
            c1 = []
            for j in range(nchunk):
                sl = pl.ds(j * CHUNK, CHUNK)
                c0.append(pltpu.async_copy(
                    emb_hbm.at[idx_v.at[pl.ds(o0 + j * CHUNK, CHUNK)]], b0.at[sl], s0))
            for j in range(nchunk):
                sl = pl.ds(j * CHUNK, CHUNK)
                c1.append(pltpu.async_copy(
                    emb_hbm.at[idx_v.at[pl.ds(o1 + j * CHUNK, CHUNK)]], b1.at[sl], s1))
            for c in c0:
                c.wait()
            r0 = pl.ds(pl.multiple_of(base + o0, 8), rows_per_buf)
            pltpu.sync_copy(b0, out_hbm.at[r0, csl])
            for c in c1:
                c.wait()
            r1 = pl.ds(pl.multiple_of(base + o1, 8), rows_per_buf)
            pltpu.sync_copy(b1, out_hbm.at[r1, csl])
            return carry

        lax.fori_loop(0, nbuf_steps // 2, body, 0)

    return k(nbr_idx2d, emb_lin)


def _leaky(x):
    return jnp.where(x >= 0, x, 0.2 * x)


BB = 16


def _tc_body(h_ref, ie_ref, nv_ref, nw_ref, adj_ref, mf_ref, al_ref,
             a4_ref, w1a_ref, w1b_ref, w2_ref, w3a_ref, w3b_ref, bias_ref,
             out_ref):
    a4 = a4_ref[...]
    BL = BB * L
    h2 = h_ref[...][:, :DIM]

    neg = jnp.full((L, L), -9e15, jnp.float32)
    att_rows = []
    for b in range(BB):
        hb = h2[b * L:(b + 1) * L]
        adjb = adj_ref[b]
        m4 = (a4[:, None, :] * hb[None, :, :]).reshape(4 * L, DIM)
        e_all = _leaky(lax.dot_general(m4, hb, (((1,), (1,)), ((), ())),
                                       preferred_element_type=jnp.float32))
        att = jnp.where(adjb == 1, e_all[0:L], neg)
        att = jnp.where(adjb == 2, e_all[L:2 * L], att)
        att = jnp.where(adjb == 3, e_all[2 * L:3 * L], att)
        att = jnp.where(adjb == 4, e_all[3 * L:4 * L], att)
        mx = jnp.max(att, axis=-1, keepdims=True)
        p = jnp.exp(att - mx)
        att = p / jnp.sum(p, axis=-1, keepdims=True)
        if b == 0:
            row = jnp.concatenate([att, jnp.zeros((L, BL - L), jnp.float32)], axis=1)
        elif b == BB - 1:
            row = jnp.concatenate([jnp.zeros((L, BL - L), jnp.float32), att], axis=1)
        else:
            row = jnp.concatenate([jnp.zeros((L, b * L), jnp.float32), att,
                                   jnp.zeros((L, BL - (b + 1) * L), jnp.float32)], axis=1)
        att_rows.append(row)
    att_bd = jnp.concatenate(att_rows, axis=0)
    h_local = jnp.dot(att_bd, h2, preferred_element_type=jnp.float32)

    mf = mf_ref[...]
    ie3 = ie_ref[...][:, :DIM].reshape(BB, L, DIM)
    sess = jnp.sum(mf.reshape(BB, L, 1) * ie3, axis=1)
    sess = sess / jnp.sum(mf, axis=-1)

    nv = nv_ref[...][:, :DIM]
    q = (nv.reshape(BB, L * S, DIM) * sess[:, None, :]).reshape(BB * L * S, DIM)
    nw2 = nw_ref[...].reshape(BB * L * S, 1)
    t = _leaky(jnp.dot(q, w1a_ref[...], preferred_element_type=jnp.float32)
               + nw2 * w1b_ref[...])
    t3 = t.reshape(BL, S, DIM)
    nv3 = nv.reshape(BL, S, DIM)
    w2 = w2_ref[...]
    ag_cols = [jnp.sum(t3[:, s, :] * w2, axis=-1, keepdims=True)
               for s in range(S)]
    ag = jnp.concatenate(ag_cols, axis=1)
    mx2 = jnp.max(ag, axis=-1, keepdims=True)
    p2 = jnp.exp(ag - mx2)
    ag = p2 / jnp.sum(p2, axis=-1, keepdims=True)
    nbr_agg = ag[:, 0:1] * nv3[:, 0, :]
    for s in range(1, S):
        nbr_agg = nbr_agg + ag[:, s:s + 1] * nv3[:, s, :]

    out = (jnp.dot(h2, w3a_ref[...], preferred_element_type=jnp.float32)
           + jnp.dot(nbr_agg, w3b_ref[...], preferred_element_type=jnp.float32)
           + bias_ref[...])
    h_global = jnp.maximum(out, 0.0)
    comb = h_local + h_global

    al2 = al_ref[...].reshape(BL, 1)
    rows = lax.broadcasted_iota(jnp.int32, (BL, 1), 0)
    tgt = al2 + (rows // L) * L
    oh = (tgt == lax.broadcasted_iota(jnp.int32, (BL, BL), 1)).astype(jnp.float32)
    final = jnp.dot(oh, comb, preferred_element_type=jnp.float32)
    out_ref[...] = final.reshape(BB, L, DIM)


def _tc_compute(h, ie, nv, nw, adj, mf, al, a4, w1a, w1b, w2, w3a, w3b, bias,
                interpret=False):
    row2 = lambda i: (i, 0)
    row3 = lambda i: (i, 0, 0)
    zero2 = lambda i: (0, 0)
    return pl.pallas_call(
        _tc_body,
        grid=(B // BB,),
        in_specs=[
            pl.BlockSpec((BB * L, PAD), row2),
            pl.BlockSpec((BB * L, PAD), row2),
            pl.BlockSpec((BB * L * S, PAD), row2),
            pl.BlockSpec((BB, L * S, 1), row3),
            pl.BlockSpec((BB, L, L), row3),
            pl.BlockSpec((BB, 1, L), row3),
            pl.BlockSpec((BB, L, 1), row3),
            pl.BlockSpec((4, DIM), zero2),
            pl.BlockSpec((DIM, DIM), zero2),
            pl.BlockSpec((1, DIM), zero2),
            pl.BlockSpec((1, DIM), zero2),
            pl.BlockSpec((DIM, DIM), zero2),
            pl.BlockSpec((DIM, DIM), zero2),
            pl.BlockSpec((1, DIM), zero2),
        ],
        out_specs=pl.BlockSpec((BB, L, DIM), row3),
        out_shape=jax.ShapeDtypeStruct((B, L, DIM), jnp.float32),
        interpret=interpret,
    )(h, ie, nv, nw, adj, mf, al, a4, w1a, w1b, w2, w3a, w3b, bias)


def kernel(input_seq, mask, items_seq, adj_seq, alias_seq, emb_table,
           adj_global, num_global, a0, a1, a2, a3, g_w1, g_w2, g_w3, g_bias):
    idx = items_seq.reshape(N_ITEMS)
    emb_lin = _emb_repack(emb_table.T).reshape(N_NODES, DIM)
    h_pad, ie_pad = _stage1(idx, input_seq.reshape(N_ITEMS), emb_lin)
    nbr_items = jnp.take(adj_global, idx, axis=0, mode="clip")
    nbr_w = jnp.take(num_global, idx, axis=0, mode="clip")
    nv_pad = _stage2(nbr_items, emb_lin)

    nw = nbr_w.reshape(B, L * S, 1)
    mf = mask.astype(jnp.float32).reshape(B, 1, L)
    al = alias_seq.reshape(B, L, 1)
    a4 = jnp.concatenate([a0, a1, a2, a3], axis=1).T
    w1a = g_w1[:DIM]
    w1b = g_w1[DIM:DIM + 1]
    w2 = g_w2.T
    w3a = g_w3[:DIM]
    w3b = g_w3[DIM:]
    bias = g_bias.reshape(1, DIM)
    return _tc_compute(h_pad, ie_pad, nv_pad, nw, adj_seq, mf, al,
                       a4, w1a, w1b, w2, w3a, w3b, bias)
